# Initial kernel scaffold; baseline (speedup 1.0000x reference)
import jax
import jax.numpy as jnp
from jax.experimental import pallas as pl


def kernel(x, embedding, W1, b1, W2, b2):
    raise NotImplementedError("write your pallas kernel here")


# speedup vs baseline: 22.7395x; 22.7395x over previous
import functools

import jax
import jax.numpy as jnp
from jax import lax
from jax.experimental import pallas as pl
from jax.experimental.pallas import tpu as pltpu
from jax.experimental.pallas import tpu_sc as plsc

VOCAB = 100000
EMBED_DIM = 64
HIDDEN_DIM = 128
OUTPUT_DIM = 5
BATCH = 16384
HIST = 200

NUM_CORES = 2
NUM_SUBCORES = 16
NUM_WORKERS = NUM_CORES * NUM_SUBCORES
ROWS_PER_WORKER = BATCH // NUM_WORKERS
IB = 16
NUM_BLOCKS = ROWS_PER_WORKER // IB
INV_HIST = 1.0 / HIST
LANES = 16
NVEC = EMBED_DIM // LANES


def _sc_mean_body(x_hbm, tab_hbm, out_hbm, idx_v, rows_v, out_v,
                  sem_i, sem0, sem1):
    wid = lax.axis_index("s") * NUM_CORES + lax.axis_index("c")
    base = wid * ROWS_PER_WORKER
    sems = (sem0, sem1)

    def fire(rb, par):
        c0 = pltpu.async_copy(
            tab_hbm.at[idx_v.at[rb, pl.ds(0, 128)]],
            rows_v.at[par, pl.ds(0, 128)], sems[par])
        c1 = pltpu.async_copy(
            tab_hbm.at[idx_v.at[rb, pl.ds(128, 72)]],
            rows_v.at[par, pl.ds(128, 72)], sems[par])
        return c0, c1

    def accumulate(rb, par):
        zero = jnp.zeros((LANES,), jnp.float32)
        accs0 = (zero,) * NVEC

        def t_body(i, accs):
            t0 = i * 8
            for dt in range(8):
                accs = tuple(
                    accs[d] + rows_v[par, t0 + dt, pl.ds(d * LANES, LANES)]
                    for d in range(NVEC))
            return accs

        accs = lax.fori_loop(0, HIST // 8, t_body, accs0)
        for d in range(NVEC):
            out_v[rb, pl.ds(d * LANES, LANES)] = accs[d] * INV_HIST

    def blk_body(blk, carry):
        rbase = base + blk * IB
        pltpu.sync_copy(x_hbm.at[pl.ds(rbase, IB), :], idx_v)
        pending = fire(0, 0)
        for rb in range(IB):
            par = rb % 2
            cur = pending
            if rb + 1 < IB:
                pending = fire(rb + 1, (rb + 1) % 2)
            cur[0].wait()
            cur[1].wait()
            accumulate(rb, par)
        pltpu.sync_copy(out_v, out_hbm.at[pl.ds(rbase, IB), :])
        return carry

    lax.fori_loop(0, NUM_BLOCKS, blk_body, 0)


@functools.partial(jax.jit, static_argnums=())
def _sc_mean(x, embedding):
    mesh = plsc.VectorSubcoreMesh(core_axis_name="c", subcore_axis_name="s")
    f = pl.kernel(
        _sc_mean_body,
        mesh=mesh,
        compiler_params=pltpu.CompilerParams(use_tc_tiling_on_sc=False),
        out_type=jax.ShapeDtypeStruct((BATCH, EMBED_DIM), jnp.float32),
        scratch_types=[
            pltpu.VMEM((IB, HIST), jnp.int32),
            pltpu.VMEM((2, HIST, EMBED_DIM), jnp.float32),
            pltpu.VMEM((IB, EMBED_DIM), jnp.float32),
            pltpu.SemaphoreType.DMA,
            pltpu.SemaphoreType.DMA,
            pltpu.SemaphoreType.DMA,
        ],
    )
    return f(x, embedding)


def _mlp_body(avg_ref, w1_ref, b1_ref, w2_ref, b2_ref, out_ref):
    h = jnp.dot(avg_ref[...], w1_ref[...],
                preferred_element_type=jnp.float32) + b1_ref[...]
    h = jnp.maximum(h, 0.0)
    out_ref[...] = jnp.dot(h, w2_ref[...],
                           preferred_element_type=jnp.float32) + b2_ref[...]


def _mlp(avg, W1, b1, W2, b2):
    bs = 2048
    grid = (BATCH // bs,)
    return pl.pallas_call(
        _mlp_body,
        grid=grid,
        in_specs=[
            pl.BlockSpec((bs, EMBED_DIM), lambda i: (i, 0)),
            pl.BlockSpec((EMBED_DIM, HIDDEN_DIM), lambda i: (0, 0)),
            pl.BlockSpec((1, HIDDEN_DIM), lambda i: (0, 0)),
            pl.BlockSpec((HIDDEN_DIM, OUTPUT_DIM), lambda i: (0, 0)),
            pl.BlockSpec((1, OUTPUT_DIM), lambda i: (0, 0)),
        ],
        out_specs=pl.BlockSpec((bs, OUTPUT_DIM), lambda i: (i, 0)),
        out_shape=jax.ShapeDtypeStruct((BATCH, OUTPUT_DIM), jnp.float32),
    )(avg, W1, b1.reshape(1, HIDDEN_DIM), W2, b2.reshape(1, OUTPUT_DIM))


def kernel(x, embedding, W1, b1, W2, b2):
    avg = _sc_mean(x, embedding)
    return _mlp(avg, W1, b1, W2, b2)
